# Initial kernel scaffold; baseline (speedup 1.0000x reference)
import jax
import jax.numpy as jnp
from jax.experimental import pallas as pl


def kernel(text, table, tune_table):
    raise NotImplementedError("write your pallas kernel here")


# speedup vs baseline: 4.0027x; 4.0027x over previous
import functools

import jax
import jax.numpy as jnp
from jax import lax
from jax.experimental import pallas as pl
from jax.experimental.pallas import tpu as pltpu
from jax.experimental.pallas import tpu_sc as plsc

VOCAB = 100000
EMB = 128
FT = 64
B = 4096
L = 200

BT = B * L
NC, NS = 2, 16
NW = NC * NS
PW = BT // NW
C = 256
KR = C // 128
NCHUNK = PW // C
IDX_ROWS_PER_W = PW // 128


def _sc_embed(idx2d, table, tune_padded):
    mesh = plsc.VectorSubcoreMesh(core_axis_name="c", subcore_axis_name="s")

    @functools.partial(
        pl.kernel,
        mesh=mesh,
        out_type=jax.ShapeDtypeStruct((BT, EMB + FT), jnp.float32),
        scratch_types=[
            pltpu.VMEM((KR, 128), jnp.int32),
            pltpu.VMEM((C, EMB), jnp.float32),
            pltpu.VMEM((C, EMB + FT), jnp.float32),
            pltpu.SemaphoreType.DMA,
        ],
    )
    def k(idx_hbm, tab_hbm, tun_hbm, out_hbm, idx_v, tun_v, cat_v, sem):
        wid = lax.axis_index("s") * NC + lax.axis_index("c")
        row_base = wid * IDX_ROWS_PER_W
        elem_base = wid * PW

        def body(i, carry):
            pltpu.sync_copy(idx_hbm.at[pl.ds(row_base + i * KR, KR)], idx_v)
            cps = []
            for j in range(KR):
                cps.append(pltpu.async_copy(
                    tab_hbm.at[idx_v.at[j]],
                    cat_v.at[pl.ds(j * 128, 128), pl.ds(0, EMB)], sem))
                cps.append(pltpu.async_copy(
                    tun_hbm.at[idx_v.at[j]],
                    tun_v.at[pl.ds(j * 128, 128)], sem))
            for cp in cps:
                cp.wait()
            def repack(kk, c2):
                for s in range(FT // 16):
                    cat_v[kk, pl.ds(EMB + s * 16, 16)] = (
                        tun_v[kk, pl.ds(s * 16, 16)])
                return c2

            lax.fori_loop(0, C, repack, 0)
            cbase = elem_base + i * C
            pltpu.sync_copy(cat_v, out_hbm.at[pl.ds(cbase, C)])
            return carry

        lax.fori_loop(0, NCHUNK, body, 0)

    return k(idx2d, table, tune_padded)


def kernel(text, table, tune_table):
    idx2d = text.reshape(BT // 128, 128)
    tune_padded = jnp.pad(tune_table, ((0, 0), (0, EMB - FT)))
    out = _sc_embed(idx2d, table, tune_padded)
    return out.reshape(B, L, EMB + FT)


# speedup vs baseline: 5.3114x; 1.3270x over previous
import functools

import jax
import jax.numpy as jnp
from jax import lax
from jax.experimental import pallas as pl
from jax.experimental.pallas import tpu as pltpu
from jax.experimental.pallas import tpu_sc as plsc

VOCAB = 100000
EMB = 128
FT = 64
B = 4096
L = 200

BT = B * L
NC, NS = 2, 16
NW = NC * NS
PW = BT // NW
C = 128
NCHUNK = PW // C


def _sc_embed(idx2d, table, tune_padded):
    mesh = plsc.VectorSubcoreMesh(core_axis_name="c", subcore_axis_name="s")

    @functools.partial(
        pl.kernel,
        mesh=mesh,
        out_type=jax.ShapeDtypeStruct((BT, EMB + FT), jnp.float32),
        scratch_types=[
            pltpu.VMEM((NCHUNK, C), jnp.int32),
            pltpu.VMEM((C, EMB), jnp.float32),
            pltpu.VMEM((C, EMB), jnp.float32),
            pltpu.VMEM((C, EMB + FT), jnp.float32),
            pltpu.VMEM((C, EMB + FT), jnp.float32),
            pltpu.SemaphoreType.DMA,
            pltpu.SemaphoreType.DMA,
            pltpu.SemaphoreType.DMA,
            pltpu.SemaphoreType.DMA,
        ],
    )
    def k(idx_hbm, tab_hbm, tun_hbm, out_hbm, idx_all,
          tun0, tun1, cat0, cat1, g0, g1, w0, w1):
        wid = lax.axis_index("s") * NC + lax.axis_index("c")
        elem_base = wid * PW
        tun = (tun0, tun1)
        cat = (cat0, cat1)
        gs = (g0, g1)
        ws = (w0, w1)

        pltpu.sync_copy(idx_hbm.at[pl.ds(wid * NCHUNK, NCHUNK)], idx_all)

        def fire_gathers(i, b):
            pltpu.async_copy(tab_hbm.at[idx_all.at[i]],
                             cat[b].at[:, pl.ds(0, EMB)], gs[b])
            pltpu.async_copy(tun_hbm.at[idx_all.at[i]], tun[b], gs[b])

        def wait_gathers(i, b):
            pltpu.make_async_copy(tab_hbm.at[idx_all.at[i]],
                                  cat[b].at[:, pl.ds(0, EMB)], gs[b]).wait()
            pltpu.make_async_copy(tun_hbm.at[idx_all.at[i]],
                                  tun[b], gs[b]).wait()

        def fire_write(i, b):
            pltpu.async_copy(cat[b], out_hbm.at[pl.ds(elem_base + i * C, C)],
                             ws[b])

        def wait_write(i, b):
            pltpu.make_async_copy(cat[b],
                                  out_hbm.at[pl.ds(elem_base + i * C, C)],
                                  ws[b]).wait()

        def repack(b):
            def rp(kk, c2):
                for u in range(4):
                    k0 = kk * 4 + u
                    for s in range(FT // 16):
                        cat[b][k0, pl.ds(EMB + s * 16, 16)] = (
                            tun[b][k0, pl.ds(s * 16, 16)])
                return c2
            lax.fori_loop(0, C // 4, rp, 0)

        fire_gathers(0, 0)

        def pair(t, carry):
            for p in (0, 1):
                i = 2 * t + p
                b = p

                @pl.when(i > 0)
                def _():
                    wait_write(i - 1, 1 - b)

                @pl.when(i < NCHUNK - 1)
                def _():
                    fire_gathers(i + 1, 1 - b)

                wait_gathers(i, b)
                repack(b)
                fire_write(i, b)
            return carry

        lax.fori_loop(0, NCHUNK // 2, pair, 0)
        wait_write(NCHUNK - 1, 1)

    return k(idx2d, table, tune_padded)


def kernel(text, table, tune_table):
    idx2d = text.reshape(BT // C, C)
    tune_padded = jnp.pad(tune_table, ((0, 0), (0, EMB - FT)))
    out = _sc_embed(idx2d, table, tune_padded)
    return out.reshape(B, L, EMB + FT)
